# Initial kernel scaffold; baseline (speedup 1.0000x reference)
import jax
import jax.numpy as jnp
from jax.experimental import pallas as pl


def kernel(x, edge_index, mlp_W, mlp_b, mlp_bn_g, mlp_bn_b, mlp_bn_m, mlp_bn_v, c1_Wg, c1_Wh, c1_bh, c1_Wcon, c1_Wdis, c1_Wself, c1_bself, c1_gateW, c1_gateb, bn1_g, bn1_b, bn1_m, bn1_v, c2_Wg, c2_Wh, c2_bh, c2_Wcon, c2_Wdis, c2_Wself, c2_bself, c2_gateW, c2_gateb, cls_W, cls_b):
    raise NotImplementedError("write your pallas kernel here")


# speedup vs baseline: 9.7243x; 9.7243x over previous
import functools

import jax
import jax.numpy as jnp
from jax import lax
from jax.experimental import pallas as pl
from jax.experimental.pallas import tpu as pltpu
from jax.experimental.pallas import tpu_sc as plsc

N = 10000
E = 320000
D = 128
H = 128
C = 40
NP = 10240
HH = 64
ET = E + N
ET_PAD = 335872
CH = 128
NB_B = ET_PAD // (32 * CH)
NB_D = ET_PAD // (16 * CH)
TSL = NP // 16

f32 = jnp.float32
i32 = jnp.int32

_mesh = plsc.VectorSubcoreMesh(core_axis_name="c", subcore_axis_name="s")
_sc_params = pltpu.CompilerParams(needs_layout_passes=False)


def _rsqrt(x):
    xi = lax.bitcast_convert_type(x, i32)
    yi = jnp.full((16,), 0x5F3759DF, i32) - lax.shift_right_arithmetic(xi, 1)
    y = lax.bitcast_convert_type(yi, f32)
    for _ in range(4):
        y = y * (1.5 - 0.5 * x * y * y)
    return y


def _edge_body(xg_hbm, a_hbm, b_hbm, row_hbm, col_hbm,
               ec_hbm, ed_hbm, scp_hbm, sdp_hbm,
               a_v, b_v, row_v, col_v, u_v, v_v, dtmp_v, ecb, edb, zc_v,
               sc_sh, sd_sh, sem1, sem2):
    cid = lax.axis_index("c")
    sid = lax.axis_index("s")
    wid = cid * 16 + sid

    pltpu.sync_copy(a_hbm, a_v)
    pltpu.sync_copy(b_hbm, b_v)

    @pl.loop(0, TSL, step=16)
    def _(i):
        zc_v[pl.ds(i, 16)] = jnp.zeros((16,), f32)

    pltpu.sync_copy(zc_v, sc_sh.at[pl.ds(sid * TSL, TSL)])
    pltpu.sync_copy(zc_v, sd_sh.at[pl.ds(sid * TSL, TSL)])
    plsc.subcore_barrier()

    base0 = wid * (NB_B * CH)
    iota16 = lax.iota(i32, 16)

    @pl.loop(0, NB_B)
    def _(ci):
        base = base0 + ci * CH
        pltpu.sync_copy(row_hbm.at[pl.ds(base, CH)], row_v)
        pltpu.sync_copy(col_hbm.at[pl.ds(base, CH)], col_v)
        cp1 = pltpu.async_copy(xg_hbm.at[row_v], u_v, sem1)
        cp2 = pltpu.async_copy(xg_hbm.at[col_v], v_v, sem2)
        cp1.wait()
        cp2.wait()

        @pl.loop(0, CH, step=16)
        def _(g0):
            for e in range(16):
                acc = jnp.zeros((16,), f32)
                for j in range(8):
                    du = u_v[g0 + e, pl.ds(j * 16, 16)] - v_v[g0 + e, pl.ds(j * 16, 16)]
                    acc = acc + du * du
                dtmp_v[e, :] = acc
            sq = jnp.zeros((16,), f32)
            for l in range(16):
                sq = sq + plsc.load_gather(dtmp_v, [iota16, jnp.full((16,), l, i32)])
            sq = jnp.maximum(sq, 1e-24)
            g = sq * _rsqrt(sq)
            rows16 = row_v[pl.ds(g0, 16)]
            cols16 = col_v[pl.ds(g0, 16)]
            z = plsc.load_gather(a_v, [rows16]) + plsc.load_gather(b_v, [cols16])
            s = 1.0 / (1.0 + jnp.exp(g) + jnp.exp(g + z))
            ec = jnp.exp(s)
            ecb[pl.ds(g0, 16)] = ec
            edb[pl.ds(g0, 16)] = 1.0 / ec

        pltpu.sync_copy(ecb, ec_hbm.at[pl.ds(base, CH)])
        pltpu.sync_copy(edb, ed_hbm.at[pl.ds(base, CH)])
        pltpu.sync_copy(ecb, sc_sh.at[row_v], add=True)
        pltpu.sync_copy(edb, sd_sh.at[row_v], add=True)

    plsc.subcore_barrier()
    sl = pl.ds(sid * TSL, TSL)
    pltpu.sync_copy(sc_sh.at[sl], scp_hbm.at[cid, sl])
    pltpu.sync_copy(sd_sh.at[sl], sdp_hbm.at[cid, sl])


_edge_kernel = functools.partial(
    pl.kernel, _edge_body, mesh=_mesh, compiler_params=_sc_params,
    out_type=[jax.ShapeDtypeStruct((ET_PAD,), f32),
              jax.ShapeDtypeStruct((ET_PAD,), f32),
              jax.ShapeDtypeStruct((2, NP), f32),
              jax.ShapeDtypeStruct((2, NP), f32)],
    scratch_types=[pltpu.VMEM((NP,), f32),
                   pltpu.VMEM((NP,), f32),
                   pltpu.VMEM((CH,), i32),
                   pltpu.VMEM((CH,), i32),
                   pltpu.VMEM((CH, H), f32),
                   pltpu.VMEM((CH, H), f32),
                   pltpu.VMEM((16, 16), f32),
                   pltpu.VMEM((CH,), f32),
                   pltpu.VMEM((CH,), f32),
                   pltpu.VMEM((TSL,), f32),
                   pltpu.VMEM_SHARED((NP,), f32),
                   pltpu.VMEM_SHARED((NP,), f32),
                   pltpu.SemaphoreType.DMA,
                   pltpu.SemaphoreType.DMA])()


def _agg_body(hh_hbm, row_hbm, col_hbm, ec_hbm, ed_hbm, rc_hbm, rd_hbm,
              accc_hbm, accd_hbm,
              r_v, row_v, col_v, ecv, w_v, hr_v, cb_v,
              acc_sh, sem1):
    cid = lax.axis_index("c")
    sid = lax.axis_index("s")

    @pl.when(cid == 0)
    def _():
        pltpu.sync_copy(rc_hbm.at[0], r_v)

    @pl.when(cid == 1)
    def _():
        pltpu.sync_copy(rd_hbm.at[0], r_v)

    @pl.loop(0, CH)
    def _(e0):
        for j in range(8):
            cb_v[e0, pl.ds(j * 16, 16)] = jnp.zeros((16,), f32)

    for k in range(TSL // CH):
        zsl = pl.ds(sid * TSL + k * CH, CH)
        pltpu.sync_copy(cb_v, acc_sh.at[zsl])
    plsc.subcore_barrier()

    base0 = sid * (NB_D * CH)

    @pl.loop(0, NB_D)
    def _(ci):
        base = base0 + ci * CH
        pltpu.sync_copy(row_hbm.at[pl.ds(base, CH)], row_v)
        pltpu.sync_copy(col_hbm.at[pl.ds(base, CH)], col_v)

        @pl.when(cid == 0)
        def _():
            pltpu.sync_copy(ec_hbm.at[pl.ds(base, CH)], ecv)

        @pl.when(cid == 1)
        def _():
            pltpu.sync_copy(ed_hbm.at[pl.ds(base, CH)], ecv)

        pltpu.async_copy(hh_hbm.at[row_v], hr_v, sem1).wait()

        @pl.loop(0, CH, step=16)
        def _(g0):
            r16 = row_v[pl.ds(g0, 16)]
            sl = pl.ds(g0, 16)
            w_v[sl] = ecv[sl] * plsc.load_gather(r_v, [r16])
            for e in range(16):
                eidx = jnp.zeros((16,), i32) + (g0 + e)
                bw = plsc.load_gather(w_v, [eidx])
                for j in range(8):
                    hv = hr_v[g0 + e, pl.ds(j * 16, 16)]
                    cb_v[g0 + e, pl.ds(j * 16, 16)] = hv * bw

        pltpu.sync_copy(cb_v, acc_sh.at[col_v], add=True)

    plsc.subcore_barrier()
    sl = pl.ds(sid * TSL, TSL)

    @pl.when(cid == 0)
    def _():
        pltpu.sync_copy(acc_sh.at[sl], accc_hbm.at[sl])

    @pl.when(cid == 1)
    def _():
        pltpu.sync_copy(acc_sh.at[sl], accd_hbm.at[sl])


_agg_kernel = functools.partial(
    pl.kernel, _agg_body, mesh=_mesh, compiler_params=_sc_params,
    out_type=[jax.ShapeDtypeStruct((NP, H), f32),
              jax.ShapeDtypeStruct((NP, H), f32)],
    scratch_types=[pltpu.VMEM((NP,), f32),
                   pltpu.VMEM((CH,), i32),
                   pltpu.VMEM((CH,), i32),
                   pltpu.VMEM((CH,), f32),
                   pltpu.VMEM((CH,), f32),
                   pltpu.VMEM((CH, H), f32),
                   pltpu.VMEM((CH, H), f32),
                   pltpu.VMEM_SHARED((NP, H), f32),
                   pltpu.SemaphoreType.DMA])()


def _recip_body(scp_ref, sdp_ref, rc_ref, rd_ref):
    rc_ref[...] = 1.0 / (scp_ref[0:1] + scp_ref[1:2] + 1e-16)
    rd_ref[...] = 1.0 / (sdp_ref[0:1] + sdp_ref[1:2] + 1e-16)


def _recip(scp, sdp):
    return pl.pallas_call(
        _recip_body,
        out_shape=[jax.ShapeDtypeStruct((1, NP), f32),
                   jax.ShapeDtypeStruct((1, NP), f32)],
    )(scp, sdp)

BR = 1000
_GRID = (N // BR,)


def _rep(shape):
    return pl.BlockSpec(shape, lambda i: tuple(0 for _ in shape))


def _rows(w):
    return pl.BlockSpec((BR, w), lambda i: (i, 0))


def _tc0_body(x_ref, WmT, mb, bng, bnb, bnm, bnv, WgT, wha, whb, bh,
              h0_ref, xg_ref, a_ref, b_ref):
    xb = x_ref[...]
    z = jnp.dot(xb, WmT[...], preferred_element_type=f32) + mb[...]
    z = (z - bnm[...]) / jnp.sqrt(bnv[...] + 1e-5) * bng[...] + bnb[...]
    h0 = jnp.maximum(z, 0.0)
    h0_ref[...] = h0
    xg = jnp.dot(h0, WgT[...], preferred_element_type=f32)
    xg_ref[...] = xg
    a_ref[...] = jnp.dot(xg, wha[...], preferred_element_type=f32)
    b_ref[...] = jnp.dot(xg, whb[...], preferred_element_type=f32) + bh[...]


def _tc0(x, WmT, mb, bng, bnb, bnm, bnv, WgT, wha, whb, bh):
    return pl.pallas_call(
        _tc0_body,
        grid=_GRID,
        in_specs=[_rows(D), _rep((D, H)), _rep((1, H)), _rep((1, H)), _rep((1, H)),
                  _rep((1, H)), _rep((1, H)), _rep((H, H)), _rep((H, 1)),
                  _rep((H, 1)), _rep((1, 1))],
        out_specs=[_rows(H), _rows(H), _rows(1), _rows(1)],
        out_shape=[jax.ShapeDtypeStruct((N, H), f32),
                   jax.ShapeDtypeStruct((N, H), f32),
                   jax.ShapeDtypeStruct((N, 1), f32),
                   jax.ShapeDtypeStruct((N, 1), f32)],
    )(x, WmT, mb, bng, bnb, bnm, bnv, WgT, wha, whb, bh)


def _gate_mix(oc, od, os, gWc, gWd, gWs, gb):
    gl = (jnp.dot(oc, gWc, preferred_element_type=f32)
          + jnp.dot(od, gWd, preferred_element_type=f32)
          + jnp.dot(os, gWs, preferred_element_type=f32) + gb)
    gm = jnp.max(gl, axis=1, keepdims=True)
    ge = jnp.exp(gl - gm)
    gw = ge / jnp.sum(ge, axis=1, keepdims=True)
    return gw[:, 0:1] * oc + gw[:, 1:2] * od + gw[:, 2:3] * os


def _tc1_body(conc_ref, disc_ref, h0_ref, WconT, WdisT, WselfT, bself,
              gWc, gWd, gWs, gb, bng, bnb, bnm, bnv, WgT, wha, whb, bh,
              h1_ref, xg_ref, a_ref, b_ref):
    h0 = h0_ref[...]
    oc = jnp.dot(conc_ref[...], WconT[...], preferred_element_type=f32)
    od = jnp.dot(disc_ref[...], WdisT[...], preferred_element_type=f32)
    os = jnp.dot(h0, WselfT[...], preferred_element_type=f32) + bself[...]
    h = _gate_mix(oc, od, os, gWc[...], gWd[...], gWs[...], gb[...])
    h = (h - bnm[...]) / jnp.sqrt(bnv[...] + 1e-5) * bng[...] + bnb[...]
    h1 = jnp.maximum(h, 0.0) + h0
    h1_ref[...] = h1
    xg = jnp.dot(h1, WgT[...], preferred_element_type=f32)
    xg_ref[...] = xg
    a_ref[...] = jnp.dot(xg, wha[...], preferred_element_type=f32)
    b_ref[...] = jnp.dot(xg, whb[...], preferred_element_type=f32) + bh[...]


def _tc1(conc, disc, h0, WconT, WdisT, WselfT, bself, gWc, gWd, gWs, gb,
         bng, bnb, bnm, bnv, WgT, wha, whb, bh):
    return pl.pallas_call(
        _tc1_body,
        grid=_GRID,
        in_specs=[_rows(H), _rows(H), _rows(H), _rep((H, H)), _rep((H, H)),
                  _rep((H, H)), _rep((1, H)), _rep((H, 3)), _rep((H, 3)),
                  _rep((H, 3)), _rep((1, 3)), _rep((1, H)), _rep((1, H)),
                  _rep((1, H)), _rep((1, H)), _rep((H, H)), _rep((H, 1)),
                  _rep((H, 1)), _rep((1, 1))],
        out_specs=[_rows(H), _rows(H), _rows(1), _rows(1)],
        out_shape=[jax.ShapeDtypeStruct((N, H), f32),
                   jax.ShapeDtypeStruct((N, H), f32),
                   jax.ShapeDtypeStruct((N, 1), f32),
                   jax.ShapeDtypeStruct((N, 1), f32)],
    )(conc, disc, h0, WconT, WdisT, WselfT, bself, gWc, gWd, gWs, gb,
      bng, bnb, bnm, bnv, WgT, wha, whb, bh)


def _tc2_body(conc_ref, disc_ref, h1_ref, WconT, WdisT, WselfT, bself,
              gWc, gWd, gWs, gb, clsWT, clsb, out_ref):
    h1 = h1_ref[...]
    oc = jnp.dot(conc_ref[...], WconT[...], preferred_element_type=f32)
    od = jnp.dot(disc_ref[...], WdisT[...], preferred_element_type=f32)
    os = jnp.dot(h1, WselfT[...], preferred_element_type=f32) + bself[...]
    h2 = _gate_mix(oc, od, os, gWc[...], gWd[...], gWs[...], gb[...]) + h1
    out_ref[...] = jnp.dot(h2, clsWT[...], preferred_element_type=f32) + clsb[...]


def _tc2(conc, disc, h1, WconT, WdisT, WselfT, bself, gWc, gWd, gWs, gb,
         clsWT, clsb):
    return pl.pallas_call(
        _tc2_body,
        grid=_GRID,
        in_specs=[_rows(H), _rows(H), _rows(H), _rep((H, H)), _rep((H, H)),
                  _rep((H, H)), _rep((1, H)), _rep((H, 3)), _rep((H, 3)),
                  _rep((H, 3)), _rep((1, 3)), _rep((H, C)), _rep((1, C))],
        out_specs=[_rows(C)],
        out_shape=[jax.ShapeDtypeStruct((N, C), f32)],
    )(conc, disc, h1, WconT, WdisT, WselfT, bself, gWc, gWd, gWs, gb, clsWT, clsb)[0]


def _conv_layer(h, xg, a, b2, rowp, colp):
    xgp = jnp.pad(xg, ((0, NP - N), (0, 0)))
    ap = jnp.pad(a.reshape(-1), (0, NP - N))
    bp = jnp.pad(b2.reshape(-1), (0, NP - N))
    ec, ed, scp, sdp = _edge_kernel(xgp, ap, bp, rowp, colp)
    rc, rd = _recip(scp, sdp)
    hp = jnp.pad(h, ((0, NP - N), (0, 0)))
    accc, accd = _agg_kernel(hp, rowp, colp, ec, ed, rc, rd)
    return accc[:N], accd[:N]


def kernel(x, edge_index, mlp_W, mlp_b, mlp_bn_g, mlp_bn_b, mlp_bn_m, mlp_bn_v,
           c1_Wg, c1_Wh, c1_bh, c1_Wcon, c1_Wdis, c1_Wself, c1_bself, c1_gateW, c1_gateb,
           bn1_g, bn1_b, bn1_m, bn1_v,
           c2_Wg, c2_Wh, c2_bh, c2_Wcon, c2_Wdis, c2_Wself, c2_bself, c2_gateW, c2_gateb,
           cls_W, cls_b):
    loops = jnp.arange(N, dtype=edge_index.dtype)
    pad_n = ET_PAD - ET
    pad_idx = (jnp.arange(pad_n, dtype=edge_index.dtype) % 16) + N
    rowp = jnp.concatenate([edge_index[0], loops, pad_idx])
    colp = jnp.concatenate([edge_index[1], loops, pad_idx])

    r1 = lambda v: v.reshape(1, -1).astype(f32)

    h0, xg1, a1, b1 = _tc0(
        x, mlp_W.T, r1(mlp_b), r1(mlp_bn_g), r1(mlp_bn_b), r1(mlp_bn_m),
        r1(mlp_bn_v), c1_Wg.T, c1_Wh[0, :H].reshape(H, 1),
        c1_Wh[0, H:].reshape(H, 1), c1_bh.reshape(1, 1))

    conc, disc = _conv_layer(h0, xg1, a1, b1, rowp, colp)

    g1 = c1_gateW.T
    h1, xg2, a2, b2 = _tc1(
        conc, disc, h0, c1_Wcon.T, c1_Wdis.T, c1_Wself.T, r1(c1_bself),
        g1[:H], g1[H:2 * H], g1[2 * H:], r1(c1_gateb),
        r1(bn1_g), r1(bn1_b), r1(bn1_m), r1(bn1_v),
        c2_Wg.T, c2_Wh[0, :H].reshape(H, 1), c2_Wh[0, H:].reshape(H, 1),
        c2_bh.reshape(1, 1))

    conc2, disc2 = _conv_layer(h1, xg2, a2, b2, rowp, colp)

    g2 = c2_gateW.T
    return _tc2(conc2, disc2, h1, c2_Wcon.T, c2_Wdis.T, c2_Wself.T,
                r1(c2_bself), g2[:H], g2[H:2 * H], g2[2 * H:], r1(c2_gateb),
                cls_W.T, r1(cls_b))


# speedup vs baseline: 16.0253x; 1.6480x over previous
import functools

import jax
import jax.numpy as jnp
from jax import lax
from jax.experimental import pallas as pl
from jax.experimental.pallas import tpu as pltpu
from jax.experimental.pallas import tpu_sc as plsc

N = 10000
E = 320000
D = 128
H = 128
C = 40
NP = 10240
HH = 64
ET = E + N
ET_PAD = 335872
CH = 128
NB_B = ET_PAD // (32 * CH)
NB_D = ET_PAD // (16 * CH)
TSL = NP // 16

f32 = jnp.float32
i32 = jnp.int32

_mesh = plsc.VectorSubcoreMesh(core_axis_name="c", subcore_axis_name="s")
_sc_params = pltpu.CompilerParams(needs_layout_passes=False)


def _rsqrt(x):
    xi = lax.bitcast_convert_type(x, i32)
    yi = jnp.full((16,), 0x5F3759DF, i32) - lax.shift_right_arithmetic(xi, 1)
    y = lax.bitcast_convert_type(yi, f32)
    for _ in range(4):
        y = y * (1.5 - 0.5 * x * y * y)
    return y


def _edge_body(xg_hbm, a_hbm, b_hbm, row_hbm, col_hbm,
               ec_hbm, ed_hbm, scp_hbm, sdp_hbm,
               a_v, b_v, row0_v, col0_v, row1_v, col1_v,
               u0_v, v0_v, u1_v, v1_v, dtmp_v, ecb, edb, ecall, edall, zc_v,
               sc_sh, sd_sh, si0, si1, sg0, sg1):
    cid = lax.axis_index("c")
    sid = lax.axis_index("s")
    wid = cid * 16 + sid

    pltpu.sync_copy(a_hbm, a_v)
    pltpu.sync_copy(b_hbm, b_v)

    @pl.loop(0, TSL, step=16)
    def _(i):
        zc_v[pl.ds(i, 16)] = jnp.zeros((16,), f32)

    pltpu.sync_copy(zc_v, sc_sh.at[pl.ds(sid * TSL, TSL)])
    pltpu.sync_copy(zc_v, sd_sh.at[pl.ds(sid * TSL, TSL)])
    plsc.subcore_barrier()

    base0 = wid * (NB_B * CH)
    iota16 = lax.iota(i32, 16)

    def issue_idx(c, row_v, col_v, sem):
        base = base0 + c * CH
        pltpu.async_copy(row_hbm.at[pl.ds(base, CH)], row_v, sem)
        pltpu.async_copy(col_hbm.at[pl.ds(base, CH)], col_v, sem)

    def drain_idx(row_v, col_v, sem):
        pltpu.make_async_copy(row_hbm.at[pl.ds(0, CH)], row_v, sem).wait()
        pltpu.make_async_copy(col_hbm.at[pl.ds(0, CH)], col_v, sem).wait()

    def issue_gather(row_v, col_v, u_v, v_v, sem):
        pltpu.async_copy(xg_hbm.at[row_v], u_v, sem)
        pltpu.async_copy(xg_hbm.at[col_v], v_v, sem)

    def drain_gather(u_v, v_v, sem):
        pltpu.make_async_copy(xg_hbm.at[pl.ds(0, CH)], u_v, sem).wait()
        pltpu.make_async_copy(xg_hbm.at[pl.ds(0, CH)], v_v, sem).wait()

    def comp(ci, row_v, col_v, u_v, v_v):
        @pl.loop(0, CH, step=16)
        def _(g0):
            for e in range(16):
                acc = jnp.zeros((16,), f32)
                for j in range(8):
                    du = u_v[g0 + e, pl.ds(j * 16, 16)] - v_v[g0 + e, pl.ds(j * 16, 16)]
                    acc = acc + du * du
                dtmp_v[e, :] = acc
            sq = jnp.zeros((16,), f32)
            for l in range(16):
                sq = sq + plsc.load_gather(dtmp_v, [iota16, jnp.full((16,), l, i32)])
            sq = jnp.maximum(sq, 1e-24)
            g = sq * _rsqrt(sq)
            rows16 = row_v[pl.ds(g0, 16)]
            cols16 = col_v[pl.ds(g0, 16)]
            z = plsc.load_gather(a_v, [rows16]) + plsc.load_gather(b_v, [cols16])
            s = 1.0 / (1.0 + jnp.exp(g) + jnp.exp(g + z))
            ec = jnp.exp(s)
            ed = 1.0 / ec
            ecb[pl.ds(g0, 16)] = ec
            edb[pl.ds(g0, 16)] = ed
            ecall[pl.ds(ci * CH + g0, 16)] = ec
            edall[pl.ds(ci * CH + g0, 16)] = ed

        pltpu.sync_copy(ecb, sc_sh.at[row_v], add=True)
        pltpu.sync_copy(edb, sd_sh.at[row_v], add=True)

    issue_idx(0, row0_v, col0_v, si0)
    drain_idx(row0_v, col0_v, si0)
    issue_gather(row0_v, col0_v, u0_v, v0_v, sg0)
    issue_idx(1, row1_v, col1_v, si1)

    @pl.loop(0, NB_B - 2, step=2)
    def _(c):
        drain_gather(u0_v, v0_v, sg0)
        drain_idx(row1_v, col1_v, si1)
        issue_gather(row1_v, col1_v, u1_v, v1_v, sg1)
        comp(c, row0_v, col0_v, u0_v, v0_v)
        issue_idx(c + 2, row0_v, col0_v, si0)

        drain_gather(u1_v, v1_v, sg1)
        drain_idx(row0_v, col0_v, si0)
        issue_gather(row0_v, col0_v, u0_v, v0_v, sg0)
        comp(c + 1, row1_v, col1_v, u1_v, v1_v)
        issue_idx(c + 3, row1_v, col1_v, si1)

    drain_gather(u0_v, v0_v, sg0)
    drain_idx(row1_v, col1_v, si1)
    issue_gather(row1_v, col1_v, u1_v, v1_v, sg1)
    comp(NB_B - 2, row0_v, col0_v, u0_v, v0_v)
    drain_gather(u1_v, v1_v, sg1)
    comp(NB_B - 1, row1_v, col1_v, u1_v, v1_v)

    pltpu.sync_copy(ecall, ec_hbm.at[pl.ds(base0, NB_B * CH)])
    pltpu.sync_copy(edall, ed_hbm.at[pl.ds(base0, NB_B * CH)])

    plsc.subcore_barrier()
    sl = pl.ds(sid * TSL, TSL)
    pltpu.sync_copy(sc_sh.at[sl], scp_hbm.at[cid, sl])
    pltpu.sync_copy(sd_sh.at[sl], sdp_hbm.at[cid, sl])


_edge_kernel = functools.partial(
    pl.kernel, _edge_body, mesh=_mesh, compiler_params=_sc_params,
    out_type=[jax.ShapeDtypeStruct((ET_PAD,), f32),
              jax.ShapeDtypeStruct((ET_PAD,), f32),
              jax.ShapeDtypeStruct((2, NP), f32),
              jax.ShapeDtypeStruct((2, NP), f32)],
    scratch_types=[pltpu.VMEM((NP,), f32),
                   pltpu.VMEM((NP,), f32),
                   pltpu.VMEM((CH,), i32),
                   pltpu.VMEM((CH,), i32),
                   pltpu.VMEM((CH,), i32),
                   pltpu.VMEM((CH,), i32),
                   pltpu.VMEM((CH, H), f32),
                   pltpu.VMEM((CH, H), f32),
                   pltpu.VMEM((CH, H), f32),
                   pltpu.VMEM((CH, H), f32),
                   pltpu.VMEM((16, 16), f32),
                   pltpu.VMEM((CH,), f32),
                   pltpu.VMEM((CH,), f32),
                   pltpu.VMEM((NB_B * CH,), f32),
                   pltpu.VMEM((NB_B * CH,), f32),
                   pltpu.VMEM((TSL,), f32),
                   pltpu.VMEM_SHARED((NP,), f32),
                   pltpu.VMEM_SHARED((NP,), f32),
                   pltpu.SemaphoreType.DMA,
                   pltpu.SemaphoreType.DMA,
                   pltpu.SemaphoreType.DMA,
                   pltpu.SemaphoreType.DMA])()


def _agg_body(hh_hbm, row_hbm, col_hbm, ec_hbm, ed_hbm, rc_hbm, rd_hbm,
              accc_hbm, accd_hbm,
              r_v, row0_v, col0_v, ec0_v, row1_v, col1_v, ec1_v, w_v,
              hr0_v, hr1_v, acc_sh, si0, si1, sg0, sg1):
    cid = lax.axis_index("c")
    sid = lax.axis_index("s")

    @pl.when(cid == 0)
    def _():
        pltpu.sync_copy(rc_hbm.at[0], r_v)

    @pl.when(cid == 1)
    def _():
        pltpu.sync_copy(rd_hbm.at[0], r_v)

    @pl.loop(0, CH)
    def _(e0):
        for j in range(8):
            hr0_v[e0, pl.ds(j * 16, 16)] = jnp.zeros((16,), f32)

    for k in range(TSL // CH):
        zsl = pl.ds(sid * TSL + k * CH, CH)
        pltpu.sync_copy(hr0_v, acc_sh.at[zsl])
    plsc.subcore_barrier()

    base0 = sid * (NB_D * CH)

    def issue_idx(c, row_v, col_v, ec_v, sem):
        base = base0 + c * CH
        pltpu.async_copy(row_hbm.at[pl.ds(base, CH)], row_v, sem)
        pltpu.async_copy(col_hbm.at[pl.ds(base, CH)], col_v, sem)

        @pl.when(cid == 0)
        def _():
            pltpu.async_copy(ec_hbm.at[pl.ds(base, CH)], ec_v, sem)

        @pl.when(cid == 1)
        def _():
            pltpu.async_copy(ed_hbm.at[pl.ds(base, CH)], ec_v, sem)

    def drain_idx(row_v, col_v, ec_v, sem):
        pltpu.make_async_copy(row_hbm.at[pl.ds(0, CH)], row_v, sem).wait()
        pltpu.make_async_copy(col_hbm.at[pl.ds(0, CH)], col_v, sem).wait()
        pltpu.make_async_copy(ec_hbm.at[pl.ds(0, CH)], ec_v, sem).wait()

    def issue_gather(row_v, hr_v, sem):
        pltpu.async_copy(hh_hbm.at[row_v], hr_v, sem)

    def drain_gather(hr_v, sem):
        pltpu.make_async_copy(hh_hbm.at[pl.ds(0, CH)], hr_v, sem).wait()

    def comp(row_v, col_v, ec_v, hr_v):
        @pl.loop(0, CH, step=16)
        def _(g0):
            r16 = row_v[pl.ds(g0, 16)]
            sl = pl.ds(g0, 16)
            w_v[sl] = ec_v[sl] * plsc.load_gather(r_v, [r16])
            for e in range(16):
                eidx = jnp.zeros((16,), i32) + (g0 + e)
                bw = plsc.load_gather(w_v, [eidx])
                for j in range(8):
                    slj = pl.ds(j * 16, 16)
                    hr_v[g0 + e, slj] = hr_v[g0 + e, slj] * bw

        pltpu.sync_copy(hr_v, acc_sh.at[col_v], add=True)

    issue_idx(0, row0_v, col0_v, ec0_v, si0)
    drain_idx(row0_v, col0_v, ec0_v, si0)
    issue_gather(row0_v, hr0_v, sg0)
    issue_idx(1, row1_v, col1_v, ec1_v, si1)

    @pl.loop(0, NB_D - 2, step=2)
    def _(c):
        drain_gather(hr0_v, sg0)
        drain_idx(row1_v, col1_v, ec1_v, si1)
        issue_gather(row1_v, hr1_v, sg1)
        comp(row0_v, col0_v, ec0_v, hr0_v)
        issue_idx(c + 2, row0_v, col0_v, ec0_v, si0)

        drain_gather(hr1_v, sg1)
        drain_idx(row0_v, col0_v, ec0_v, si0)
        issue_gather(row0_v, hr0_v, sg0)
        comp(row1_v, col1_v, ec1_v, hr1_v)
        issue_idx(c + 3, row1_v, col1_v, ec1_v, si1)

    drain_gather(hr0_v, sg0)
    drain_idx(row1_v, col1_v, ec1_v, si1)
    issue_gather(row1_v, hr1_v, sg1)
    comp(row0_v, col0_v, ec0_v, hr0_v)
    drain_gather(hr1_v, sg1)
    comp(row1_v, col1_v, ec1_v, hr1_v)

    plsc.subcore_barrier()
    sl = pl.ds(sid * TSL, TSL)

    @pl.when(cid == 0)
    def _():
        pltpu.sync_copy(acc_sh.at[sl], accc_hbm.at[sl])

    @pl.when(cid == 1)
    def _():
        pltpu.sync_copy(acc_sh.at[sl], accd_hbm.at[sl])


_agg_kernel = functools.partial(
    pl.kernel, _agg_body, mesh=_mesh, compiler_params=_sc_params,
    out_type=[jax.ShapeDtypeStruct((NP, H), f32),
              jax.ShapeDtypeStruct((NP, H), f32)],
    scratch_types=[pltpu.VMEM((NP,), f32),
                   pltpu.VMEM((CH,), i32),
                   pltpu.VMEM((CH,), i32),
                   pltpu.VMEM((CH,), f32),
                   pltpu.VMEM((CH,), i32),
                   pltpu.VMEM((CH,), i32),
                   pltpu.VMEM((CH,), f32),
                   pltpu.VMEM((CH,), f32),
                   pltpu.VMEM((CH, H), f32),
                   pltpu.VMEM((CH, H), f32),
                   pltpu.VMEM_SHARED((NP, H), f32),
                   pltpu.SemaphoreType.DMA,
                   pltpu.SemaphoreType.DMA,
                   pltpu.SemaphoreType.DMA,
                   pltpu.SemaphoreType.DMA])()


def _recip_body(scp_ref, sdp_ref, rc_ref, rd_ref):
    rc_ref[...] = 1.0 / (scp_ref[0:1] + scp_ref[1:2] + 1e-16)
    rd_ref[...] = 1.0 / (sdp_ref[0:1] + sdp_ref[1:2] + 1e-16)


def _recip(scp, sdp):
    return pl.pallas_call(
        _recip_body,
        out_shape=[jax.ShapeDtypeStruct((1, NP), f32),
                   jax.ShapeDtypeStruct((1, NP), f32)],
    )(scp, sdp)

BR = 1000
_GRID = (N // BR,)


def _rep(shape):
    return pl.BlockSpec(shape, lambda i: tuple(0 for _ in shape))


def _rows(w):
    return pl.BlockSpec((BR, w), lambda i: (i, 0))


def _tc0_body(x_ref, WmT, mb, bng, bnb, bnm, bnv, WgT, wha, whb, bh,
              h0_ref, xg_ref, a_ref, b_ref):
    xb = x_ref[...]
    z = jnp.dot(xb, WmT[...], preferred_element_type=f32) + mb[...]
    z = (z - bnm[...]) / jnp.sqrt(bnv[...] + 1e-5) * bng[...] + bnb[...]
    h0 = jnp.maximum(z, 0.0)
    h0_ref[...] = h0
    xg = jnp.dot(h0, WgT[...], preferred_element_type=f32)
    xg_ref[...] = xg
    a_ref[...] = jnp.dot(xg, wha[...], preferred_element_type=f32)
    b_ref[...] = jnp.dot(xg, whb[...], preferred_element_type=f32) + bh[...]


def _tc0(x, WmT, mb, bng, bnb, bnm, bnv, WgT, wha, whb, bh):
    return pl.pallas_call(
        _tc0_body,
        grid=_GRID,
        in_specs=[_rows(D), _rep((D, H)), _rep((1, H)), _rep((1, H)), _rep((1, H)),
                  _rep((1, H)), _rep((1, H)), _rep((H, H)), _rep((H, 1)),
                  _rep((H, 1)), _rep((1, 1))],
        out_specs=[_rows(H), _rows(H), _rows(1), _rows(1)],
        out_shape=[jax.ShapeDtypeStruct((N, H), f32),
                   jax.ShapeDtypeStruct((N, H), f32),
                   jax.ShapeDtypeStruct((N, 1), f32),
                   jax.ShapeDtypeStruct((N, 1), f32)],
    )(x, WmT, mb, bng, bnb, bnm, bnv, WgT, wha, whb, bh)


def _gate_mix(oc, od, os, gWc, gWd, gWs, gb):
    gl = (jnp.dot(oc, gWc, preferred_element_type=f32)
          + jnp.dot(od, gWd, preferred_element_type=f32)
          + jnp.dot(os, gWs, preferred_element_type=f32) + gb)
    gm = jnp.max(gl, axis=1, keepdims=True)
    ge = jnp.exp(gl - gm)
    gw = ge / jnp.sum(ge, axis=1, keepdims=True)
    return gw[:, 0:1] * oc + gw[:, 1:2] * od + gw[:, 2:3] * os


def _tc1_body(conc_ref, disc_ref, h0_ref, WconT, WdisT, WselfT, bself,
              gWc, gWd, gWs, gb, bng, bnb, bnm, bnv, WgT, wha, whb, bh,
              h1_ref, xg_ref, a_ref, b_ref):
    h0 = h0_ref[...]
    oc = jnp.dot(conc_ref[...], WconT[...], preferred_element_type=f32)
    od = jnp.dot(disc_ref[...], WdisT[...], preferred_element_type=f32)
    os = jnp.dot(h0, WselfT[...], preferred_element_type=f32) + bself[...]
    h = _gate_mix(oc, od, os, gWc[...], gWd[...], gWs[...], gb[...])
    h = (h - bnm[...]) / jnp.sqrt(bnv[...] + 1e-5) * bng[...] + bnb[...]
    h1 = jnp.maximum(h, 0.0) + h0
    h1_ref[...] = h1
    xg = jnp.dot(h1, WgT[...], preferred_element_type=f32)
    xg_ref[...] = xg
    a_ref[...] = jnp.dot(xg, wha[...], preferred_element_type=f32)
    b_ref[...] = jnp.dot(xg, whb[...], preferred_element_type=f32) + bh[...]


def _tc1(conc, disc, h0, WconT, WdisT, WselfT, bself, gWc, gWd, gWs, gb,
         bng, bnb, bnm, bnv, WgT, wha, whb, bh):
    return pl.pallas_call(
        _tc1_body,
        grid=_GRID,
        in_specs=[_rows(H), _rows(H), _rows(H), _rep((H, H)), _rep((H, H)),
                  _rep((H, H)), _rep((1, H)), _rep((H, 3)), _rep((H, 3)),
                  _rep((H, 3)), _rep((1, 3)), _rep((1, H)), _rep((1, H)),
                  _rep((1, H)), _rep((1, H)), _rep((H, H)), _rep((H, 1)),
                  _rep((H, 1)), _rep((1, 1))],
        out_specs=[_rows(H), _rows(H), _rows(1), _rows(1)],
        out_shape=[jax.ShapeDtypeStruct((N, H), f32),
                   jax.ShapeDtypeStruct((N, H), f32),
                   jax.ShapeDtypeStruct((N, 1), f32),
                   jax.ShapeDtypeStruct((N, 1), f32)],
    )(conc, disc, h0, WconT, WdisT, WselfT, bself, gWc, gWd, gWs, gb,
      bng, bnb, bnm, bnv, WgT, wha, whb, bh)


def _tc2_body(conc_ref, disc_ref, h1_ref, WconT, WdisT, WselfT, bself,
              gWc, gWd, gWs, gb, clsWT, clsb, out_ref):
    h1 = h1_ref[...]
    oc = jnp.dot(conc_ref[...], WconT[...], preferred_element_type=f32)
    od = jnp.dot(disc_ref[...], WdisT[...], preferred_element_type=f32)
    os = jnp.dot(h1, WselfT[...], preferred_element_type=f32) + bself[...]
    h2 = _gate_mix(oc, od, os, gWc[...], gWd[...], gWs[...], gb[...]) + h1
    out_ref[...] = jnp.dot(h2, clsWT[...], preferred_element_type=f32) + clsb[...]


def _tc2(conc, disc, h1, WconT, WdisT, WselfT, bself, gWc, gWd, gWs, gb,
         clsWT, clsb):
    return pl.pallas_call(
        _tc2_body,
        grid=_GRID,
        in_specs=[_rows(H), _rows(H), _rows(H), _rep((H, H)), _rep((H, H)),
                  _rep((H, H)), _rep((1, H)), _rep((H, 3)), _rep((H, 3)),
                  _rep((H, 3)), _rep((1, 3)), _rep((H, C)), _rep((1, C))],
        out_specs=[_rows(C)],
        out_shape=[jax.ShapeDtypeStruct((N, C), f32)],
    )(conc, disc, h1, WconT, WdisT, WselfT, bself, gWc, gWd, gWs, gb, clsWT, clsb)[0]


def _conv_layer(h, xg, a, b2, rowp, colp):
    xgp = jnp.pad(xg, ((0, NP - N), (0, 0)))
    ap = jnp.pad(a.reshape(-1), (0, NP - N))
    bp = jnp.pad(b2.reshape(-1), (0, NP - N))
    ec, ed, scp, sdp = _edge_kernel(xgp, ap, bp, rowp, colp)
    rc, rd = _recip(scp, sdp)
    hp = jnp.pad(h, ((0, NP - N), (0, 0)))
    accc, accd = _agg_kernel(hp, rowp, colp, ec, ed, rc, rd)
    return accc[:N], accd[:N]


def kernel(x, edge_index, mlp_W, mlp_b, mlp_bn_g, mlp_bn_b, mlp_bn_m, mlp_bn_v,
           c1_Wg, c1_Wh, c1_bh, c1_Wcon, c1_Wdis, c1_Wself, c1_bself, c1_gateW, c1_gateb,
           bn1_g, bn1_b, bn1_m, bn1_v,
           c2_Wg, c2_Wh, c2_bh, c2_Wcon, c2_Wdis, c2_Wself, c2_bself, c2_gateW, c2_gateb,
           cls_W, cls_b):
    loops = jnp.arange(N, dtype=edge_index.dtype)
    pad_n = ET_PAD - ET
    pad_idx = (jnp.arange(pad_n, dtype=edge_index.dtype) % 16) + N
    rowp = jnp.concatenate([edge_index[0], loops, pad_idx])
    colp = jnp.concatenate([edge_index[1], loops, pad_idx])

    r1 = lambda v: v.reshape(1, -1).astype(f32)

    h0, xg1, a1, b1 = _tc0(
        x, mlp_W.T, r1(mlp_b), r1(mlp_bn_g), r1(mlp_bn_b), r1(mlp_bn_m),
        r1(mlp_bn_v), c1_Wg.T, c1_Wh[0, :H].reshape(H, 1),
        c1_Wh[0, H:].reshape(H, 1), c1_bh.reshape(1, 1))

    conc, disc = _conv_layer(h0, xg1, a1, b1, rowp, colp)

    g1 = c1_gateW.T
    h1, xg2, a2, b2 = _tc1(
        conc, disc, h0, c1_Wcon.T, c1_Wdis.T, c1_Wself.T, r1(c1_bself),
        g1[:H], g1[H:2 * H], g1[2 * H:], r1(c1_gateb),
        r1(bn1_g), r1(bn1_b), r1(bn1_m), r1(bn1_v),
        c2_Wg.T, c2_Wh[0, :H].reshape(H, 1), c2_Wh[0, H:].reshape(H, 1),
        c2_bh.reshape(1, 1))

    conc2, disc2 = _conv_layer(h1, xg2, a2, b2, rowp, colp)

    g2 = c2_gateW.T
    return _tc2(conc2, disc2, h1, c2_Wcon.T, c2_Wdis.T, c2_Wself.T,
                r1(c2_bself), g2[:H], g2[H:2 * H], g2[2 * H:], r1(c2_gateb),
                cls_W.T, r1(cls_b))


# speedup vs baseline: 17.8416x; 1.1133x over previous
import functools

import jax
import jax.numpy as jnp
from jax import lax
from jax.experimental import pallas as pl
from jax.experimental.pallas import tpu as pltpu
from jax.experimental.pallas import tpu_sc as plsc

N = 10000
E = 320000
D = 128
H = 128
C = 40
NP = 10240
HH = 64
ET = E + N
ET_PAD = 335872
CH = 128
NB_B = ET_PAD // (32 * CH)
NB_D = ET_PAD // (16 * CH)
TSL = NP // 16

f32 = jnp.float32
i32 = jnp.int32

_mesh = plsc.VectorSubcoreMesh(core_axis_name="c", subcore_axis_name="s")
_sc_params = pltpu.CompilerParams(needs_layout_passes=False)


def _rsqrt(x):
    xi = lax.bitcast_convert_type(x, i32)
    yi = jnp.full((16,), 0x5F3759DF, i32) - lax.shift_right_arithmetic(xi, 1)
    y = lax.bitcast_convert_type(yi, f32)
    for _ in range(4):
        y = y * (1.5 - 0.5 * x * y * y)
    return y


def _edge_body(xg_hbm, a_hbm, b_hbm, nrm_hbm, row_hbm, col_hbm,
               ec_hbm, ed_hbm, scp_hbm, sdp_hbm,
               a_v, b_v, nrm_v, row0_v, col0_v, row1_v, col1_v,
               u0_v, v0_v, u1_v, v1_v, dtmp_v, ecb, edb, ecall, edall, zc_v,
               sc_sh, sd_sh, si0, si1, sg0, sg1):
    cid = lax.axis_index("c")
    sid = lax.axis_index("s")
    wid = cid * 16 + sid

    pltpu.sync_copy(a_hbm, a_v)
    pltpu.sync_copy(b_hbm, b_v)
    pltpu.sync_copy(nrm_hbm, nrm_v)

    @pl.loop(0, TSL, step=16)
    def _(i):
        zc_v[pl.ds(i, 16)] = jnp.zeros((16,), f32)

    pltpu.sync_copy(zc_v, sc_sh.at[pl.ds(sid * TSL, TSL)])
    pltpu.sync_copy(zc_v, sd_sh.at[pl.ds(sid * TSL, TSL)])
    plsc.subcore_barrier()

    base0 = wid * (NB_B * CH)
    iota16 = lax.iota(i32, 16)

    def issue_idx(c, row_v, col_v, sem):
        base = base0 + c * CH
        pltpu.async_copy(row_hbm.at[pl.ds(base, CH)], row_v, sem)
        pltpu.async_copy(col_hbm.at[pl.ds(base, CH)], col_v, sem)

    def drain_idx(row_v, col_v, sem):
        pltpu.make_async_copy(row_hbm.at[pl.ds(0, CH)], row_v, sem).wait()
        pltpu.make_async_copy(col_hbm.at[pl.ds(0, CH)], col_v, sem).wait()

    def issue_gather(row_v, col_v, u_v, v_v, sem):
        pltpu.async_copy(xg_hbm.at[row_v], u_v, sem)
        pltpu.async_copy(xg_hbm.at[col_v], v_v, sem)

    def drain_gather(u_v, v_v, sem):
        pltpu.make_async_copy(xg_hbm.at[pl.ds(0, CH)], u_v, sem).wait()
        pltpu.make_async_copy(xg_hbm.at[pl.ds(0, CH)], v_v, sem).wait()

    def comp(ci, row_v, col_v, u_v, v_v):
        @pl.loop(0, CH, step=16)
        def _(g0):
            for e in range(16):
                acc = jnp.zeros((16,), f32)
                for j in range(8):
                    acc = acc + (u_v[g0 + e, pl.ds(j * 16, 16)]
                                 * v_v[g0 + e, pl.ds(j * 16, 16)])
                dtmp_v[e, :] = acc
            dot = jnp.zeros((16,), f32)
            for l in range(16):
                dot = dot + plsc.load_gather(dtmp_v, [iota16, jnp.full((16,), l, i32)])
            rows16 = row_v[pl.ds(g0, 16)]
            cols16 = col_v[pl.ds(g0, 16)]
            sq = (plsc.load_gather(nrm_v, [rows16])
                  + plsc.load_gather(nrm_v, [cols16]) - 2.0 * dot)
            sq = jnp.maximum(sq, 1e-24)
            g = sq * _rsqrt(sq)
            g = jnp.where(rows16 == cols16, 0.0, g)
            z = plsc.load_gather(a_v, [rows16]) + plsc.load_gather(b_v, [cols16])
            s = 1.0 / (1.0 + jnp.exp(g) + jnp.exp(g + z))
            ec = jnp.exp(s)
            ed = 1.0 / ec
            ecb[pl.ds(g0, 16)] = ec
            edb[pl.ds(g0, 16)] = ed
            ecall[pl.ds(ci * CH + g0, 16)] = ec
            edall[pl.ds(ci * CH + g0, 16)] = ed

        pltpu.sync_copy(ecb, sc_sh.at[row_v], add=True)
        pltpu.sync_copy(edb, sd_sh.at[row_v], add=True)

    issue_idx(0, row0_v, col0_v, si0)
    drain_idx(row0_v, col0_v, si0)
    issue_gather(row0_v, col0_v, u0_v, v0_v, sg0)
    issue_idx(1, row1_v, col1_v, si1)

    @pl.loop(0, NB_B - 2, step=2)
    def _(c):
        drain_gather(u0_v, v0_v, sg0)
        drain_idx(row1_v, col1_v, si1)
        issue_gather(row1_v, col1_v, u1_v, v1_v, sg1)
        comp(c, row0_v, col0_v, u0_v, v0_v)
        issue_idx(c + 2, row0_v, col0_v, si0)

        drain_gather(u1_v, v1_v, sg1)
        drain_idx(row0_v, col0_v, si0)
        issue_gather(row0_v, col0_v, u0_v, v0_v, sg0)
        comp(c + 1, row1_v, col1_v, u1_v, v1_v)
        issue_idx(c + 3, row1_v, col1_v, si1)

    drain_gather(u0_v, v0_v, sg0)
    drain_idx(row1_v, col1_v, si1)
    issue_gather(row1_v, col1_v, u1_v, v1_v, sg1)
    comp(NB_B - 2, row0_v, col0_v, u0_v, v0_v)
    drain_gather(u1_v, v1_v, sg1)
    comp(NB_B - 1, row1_v, col1_v, u1_v, v1_v)

    pltpu.sync_copy(ecall, ec_hbm.at[pl.ds(base0, NB_B * CH)])
    pltpu.sync_copy(edall, ed_hbm.at[pl.ds(base0, NB_B * CH)])

    plsc.subcore_barrier()
    sl = pl.ds(sid * TSL, TSL)
    pltpu.sync_copy(sc_sh.at[sl], scp_hbm.at[cid, sl])
    pltpu.sync_copy(sd_sh.at[sl], sdp_hbm.at[cid, sl])


_edge_kernel = functools.partial(
    pl.kernel, _edge_body, mesh=_mesh, compiler_params=_sc_params,
    out_type=[jax.ShapeDtypeStruct((ET_PAD,), f32),
              jax.ShapeDtypeStruct((ET_PAD,), f32),
              jax.ShapeDtypeStruct((2, NP), f32),
              jax.ShapeDtypeStruct((2, NP), f32)],
    scratch_types=[pltpu.VMEM((NP,), f32),
                   pltpu.VMEM((NP,), f32),
                   pltpu.VMEM((NP,), f32),
                   pltpu.VMEM((CH,), i32),
                   pltpu.VMEM((CH,), i32),
                   pltpu.VMEM((CH,), i32),
                   pltpu.VMEM((CH,), i32),
                   pltpu.VMEM((CH, H), f32),
                   pltpu.VMEM((CH, H), f32),
                   pltpu.VMEM((CH, H), f32),
                   pltpu.VMEM((CH, H), f32),
                   pltpu.VMEM((16, 16), f32),
                   pltpu.VMEM((CH,), f32),
                   pltpu.VMEM((CH,), f32),
                   pltpu.VMEM((NB_B * CH,), f32),
                   pltpu.VMEM((NB_B * CH,), f32),
                   pltpu.VMEM((TSL,), f32),
                   pltpu.VMEM_SHARED((NP,), f32),
                   pltpu.VMEM_SHARED((NP,), f32),
                   pltpu.SemaphoreType.DMA,
                   pltpu.SemaphoreType.DMA,
                   pltpu.SemaphoreType.DMA,
                   pltpu.SemaphoreType.DMA])()


def _agg_body(hh_hbm, row_hbm, col_hbm, ec_hbm, ed_hbm, rc_hbm, rd_hbm,
              accc_hbm, accd_hbm,
              r_v, row0_v, col0_v, ec0_v, row1_v, col1_v, ec1_v, w_v,
              colsc0_v, colsc1_v, hr0_v, hr1_v, acc_sh,
              si0, si1, sg0, sg1, ss0, ss1):
    cid = lax.axis_index("c")
    sid = lax.axis_index("s")

    @pl.when(cid == 0)
    def _():
        pltpu.sync_copy(rc_hbm.at[0], r_v)

    @pl.when(cid == 1)
    def _():
        pltpu.sync_copy(rd_hbm.at[0], r_v)

    @pl.loop(0, CH)
    def _(e0):
        for j in range(8):
            hr1_v[e0, pl.ds(j * 16, 16)] = jnp.zeros((16,), f32)

    for k in range(TSL // CH):
        zsl = pl.ds(sid * TSL + k * CH, CH)
        pltpu.sync_copy(hr1_v, acc_sh.at[zsl])
    plsc.subcore_barrier()

    base0 = sid * (NB_D * CH)

    def issue_idx(c, row_v, col_v, ec_v, sem):
        base = base0 + c * CH
        pltpu.async_copy(row_hbm.at[pl.ds(base, CH)], row_v, sem)
        pltpu.async_copy(col_hbm.at[pl.ds(base, CH)], col_v, sem)

        @pl.when(cid == 0)
        def _():
            pltpu.async_copy(ec_hbm.at[pl.ds(base, CH)], ec_v, sem)

        @pl.when(cid == 1)
        def _():
            pltpu.async_copy(ed_hbm.at[pl.ds(base, CH)], ec_v, sem)

    def drain_idx(row_v, col_v, ec_v, sem):
        pltpu.make_async_copy(row_hbm.at[pl.ds(0, CH)], row_v, sem).wait()
        pltpu.make_async_copy(col_hbm.at[pl.ds(0, CH)], col_v, sem).wait()
        pltpu.make_async_copy(ec_hbm.at[pl.ds(0, CH)], ec_v, sem).wait()

    def issue_gather(row_v, hr_v, sem):
        pltpu.async_copy(hh_hbm.at[row_v], hr_v, sem)

    def drain_gather(hr_v, sem):
        pltpu.make_async_copy(hh_hbm.at[pl.ds(0, CH)], hr_v, sem).wait()

    def drain_scatter(hr_v, sem):
        pltpu.make_async_copy(hh_hbm.at[pl.ds(0, CH)], hr_v, sem).wait()

    def comp(row_v, col_v, colsc_v, ec_v, hr_v, ssem):
        @pl.loop(0, CH, step=16)
        def _(g0):
            r16 = row_v[pl.ds(g0, 16)]
            sl = pl.ds(g0, 16)
            w_v[sl] = ec_v[sl] * plsc.load_gather(r_v, [r16])
            colsc_v[sl] = col_v[sl]
            for e in range(16):
                eidx = jnp.zeros((16,), i32) + (g0 + e)
                bw = plsc.load_gather(w_v, [eidx])
                for j in range(8):
                    slj = pl.ds(j * 16, 16)
                    hr_v[g0 + e, slj] = hr_v[g0 + e, slj] * bw

        pltpu.async_copy(hr_v, acc_sh.at[colsc_v], ssem, add=True)

    issue_idx(0, row0_v, col0_v, ec0_v, si0)
    drain_idx(row0_v, col0_v, ec0_v, si0)
    pltpu.async_copy(hr1_v, acc_sh.at[row0_v], ss1, add=True)
    issue_gather(row0_v, hr0_v, sg0)
    issue_idx(1, row1_v, col1_v, ec1_v, si1)

    @pl.loop(0, NB_D - 2, step=2)
    def _(c):
        drain_gather(hr0_v, sg0)
        drain_idx(row1_v, col1_v, ec1_v, si1)
        drain_scatter(hr1_v, ss1)
        issue_gather(row1_v, hr1_v, sg1)
        comp(row0_v, col0_v, colsc0_v, ec0_v, hr0_v, ss0)
        issue_idx(c + 2, row0_v, col0_v, ec0_v, si0)

        drain_gather(hr1_v, sg1)
        drain_idx(row0_v, col0_v, ec0_v, si0)
        drain_scatter(hr0_v, ss0)
        issue_gather(row0_v, hr0_v, sg0)
        comp(row1_v, col1_v, colsc1_v, ec1_v, hr1_v, ss1)
        issue_idx(c + 3, row1_v, col1_v, ec1_v, si1)

    drain_gather(hr0_v, sg0)
    drain_idx(row1_v, col1_v, ec1_v, si1)
    drain_scatter(hr1_v, ss1)
    issue_gather(row1_v, hr1_v, sg1)
    comp(row0_v, col0_v, colsc0_v, ec0_v, hr0_v, ss0)
    drain_gather(hr1_v, sg1)
    drain_scatter(hr0_v, ss0)
    comp(row1_v, col1_v, colsc1_v, ec1_v, hr1_v, ss1)
    drain_scatter(hr1_v, ss1)

    plsc.subcore_barrier()
    sl = pl.ds(sid * TSL, TSL)

    @pl.when(cid == 0)
    def _():
        pltpu.sync_copy(acc_sh.at[sl], accc_hbm.at[sl])

    @pl.when(cid == 1)
    def _():
        pltpu.sync_copy(acc_sh.at[sl], accd_hbm.at[sl])


_agg_kernel = functools.partial(
    pl.kernel, _agg_body, mesh=_mesh, compiler_params=_sc_params,
    out_type=[jax.ShapeDtypeStruct((NP, H), f32),
              jax.ShapeDtypeStruct((NP, H), f32)],
    scratch_types=[pltpu.VMEM((NP,), f32),
                   pltpu.VMEM((CH,), i32),
                   pltpu.VMEM((CH,), i32),
                   pltpu.VMEM((CH,), f32),
                   pltpu.VMEM((CH,), i32),
                   pltpu.VMEM((CH,), i32),
                   pltpu.VMEM((CH,), f32),
                   pltpu.VMEM((CH,), f32),
                   pltpu.VMEM((CH,), i32),
                   pltpu.VMEM((CH,), i32),
                   pltpu.VMEM((CH, H), f32),
                   pltpu.VMEM((CH, H), f32),
                   pltpu.VMEM_SHARED((NP, H), f32),
                   pltpu.SemaphoreType.DMA,
                   pltpu.SemaphoreType.DMA,
                   pltpu.SemaphoreType.DMA,
                   pltpu.SemaphoreType.DMA,
                   pltpu.SemaphoreType.DMA,
                   pltpu.SemaphoreType.DMA])()


def _recip_body(scp_ref, sdp_ref, rc_ref, rd_ref):
    rc_ref[...] = 1.0 / (scp_ref[0:1] + scp_ref[1:2] + 1e-16)
    rd_ref[...] = 1.0 / (sdp_ref[0:1] + sdp_ref[1:2] + 1e-16)


def _recip(scp, sdp):
    return pl.pallas_call(
        _recip_body,
        out_shape=[jax.ShapeDtypeStruct((1, NP), f32),
                   jax.ShapeDtypeStruct((1, NP), f32)],
    )(scp, sdp)

BR = 1000
_GRID = (N // BR,)


def _rep(shape):
    return pl.BlockSpec(shape, lambda i: tuple(0 for _ in shape))


def _rows(w):
    return pl.BlockSpec((BR, w), lambda i: (i, 0))


def _tc0_body(x_ref, WmT, mb, bng, bnb, bnm, bnv, WgT, wha, whb, bh,
              h0_ref, xg_ref, a_ref, b_ref, nrm_ref):
    xb = x_ref[...]
    z = jnp.dot(xb, WmT[...], preferred_element_type=f32) + mb[...]
    z = (z - bnm[...]) / jnp.sqrt(bnv[...] + 1e-5) * bng[...] + bnb[...]
    h0 = jnp.maximum(z, 0.0)
    h0_ref[...] = h0
    xg = jnp.dot(h0, WgT[...], preferred_element_type=f32)
    xg_ref[...] = xg
    a_ref[...] = jnp.dot(xg, wha[...], preferred_element_type=f32)
    b_ref[...] = jnp.dot(xg, whb[...], preferred_element_type=f32) + bh[...]
    nrm_ref[...] = jnp.sum(xg * xg, axis=1, keepdims=True)


def _tc0(x, WmT, mb, bng, bnb, bnm, bnv, WgT, wha, whb, bh):
    return pl.pallas_call(
        _tc0_body,
        grid=_GRID,
        in_specs=[_rows(D), _rep((D, H)), _rep((1, H)), _rep((1, H)), _rep((1, H)),
                  _rep((1, H)), _rep((1, H)), _rep((H, H)), _rep((H, 1)),
                  _rep((H, 1)), _rep((1, 1))],
        out_specs=[_rows(H), _rows(H), _rows(1), _rows(1), _rows(1)],
        out_shape=[jax.ShapeDtypeStruct((N, H), f32),
                   jax.ShapeDtypeStruct((N, H), f32),
                   jax.ShapeDtypeStruct((N, 1), f32),
                   jax.ShapeDtypeStruct((N, 1), f32),
                   jax.ShapeDtypeStruct((N, 1), f32)],
    )(x, WmT, mb, bng, bnb, bnm, bnv, WgT, wha, whb, bh)


def _gate_mix(oc, od, os, gWc, gWd, gWs, gb):
    gl = (jnp.dot(oc, gWc, preferred_element_type=f32)
          + jnp.dot(od, gWd, preferred_element_type=f32)
          + jnp.dot(os, gWs, preferred_element_type=f32) + gb)
    gm = jnp.max(gl, axis=1, keepdims=True)
    ge = jnp.exp(gl - gm)
    gw = ge / jnp.sum(ge, axis=1, keepdims=True)
    return gw[:, 0:1] * oc + gw[:, 1:2] * od + gw[:, 2:3] * os


def _tc1_body(conc_ref, disc_ref, h0_ref, WconT, WdisT, WselfT, bself,
              gWc, gWd, gWs, gb, bng, bnb, bnm, bnv, WgT, wha, whb, bh,
              h1_ref, xg_ref, a_ref, b_ref, nrm_ref):
    h0 = h0_ref[...]
    oc = jnp.dot(conc_ref[...], WconT[...], preferred_element_type=f32)
    od = jnp.dot(disc_ref[...], WdisT[...], preferred_element_type=f32)
    os = jnp.dot(h0, WselfT[...], preferred_element_type=f32) + bself[...]
    h = _gate_mix(oc, od, os, gWc[...], gWd[...], gWs[...], gb[...])
    h = (h - bnm[...]) / jnp.sqrt(bnv[...] + 1e-5) * bng[...] + bnb[...]
    h1 = jnp.maximum(h, 0.0) + h0
    h1_ref[...] = h1
    xg = jnp.dot(h1, WgT[...], preferred_element_type=f32)
    xg_ref[...] = xg
    a_ref[...] = jnp.dot(xg, wha[...], preferred_element_type=f32)
    b_ref[...] = jnp.dot(xg, whb[...], preferred_element_type=f32) + bh[...]
    nrm_ref[...] = jnp.sum(xg * xg, axis=1, keepdims=True)


def _tc1(conc, disc, h0, WconT, WdisT, WselfT, bself, gWc, gWd, gWs, gb,
         bng, bnb, bnm, bnv, WgT, wha, whb, bh):
    return pl.pallas_call(
        _tc1_body,
        grid=_GRID,
        in_specs=[_rows(H), _rows(H), _rows(H), _rep((H, H)), _rep((H, H)),
                  _rep((H, H)), _rep((1, H)), _rep((H, 3)), _rep((H, 3)),
                  _rep((H, 3)), _rep((1, 3)), _rep((1, H)), _rep((1, H)),
                  _rep((1, H)), _rep((1, H)), _rep((H, H)), _rep((H, 1)),
                  _rep((H, 1)), _rep((1, 1))],
        out_specs=[_rows(H), _rows(H), _rows(1), _rows(1), _rows(1)],
        out_shape=[jax.ShapeDtypeStruct((N, H), f32),
                   jax.ShapeDtypeStruct((N, H), f32),
                   jax.ShapeDtypeStruct((N, 1), f32),
                   jax.ShapeDtypeStruct((N, 1), f32),
                   jax.ShapeDtypeStruct((N, 1), f32)],
    )(conc, disc, h0, WconT, WdisT, WselfT, bself, gWc, gWd, gWs, gb,
      bng, bnb, bnm, bnv, WgT, wha, whb, bh)


def _tc2_body(conc_ref, disc_ref, h1_ref, WconT, WdisT, WselfT, bself,
              gWc, gWd, gWs, gb, clsWT, clsb, out_ref):
    h1 = h1_ref[...]
    oc = jnp.dot(conc_ref[...], WconT[...], preferred_element_type=f32)
    od = jnp.dot(disc_ref[...], WdisT[...], preferred_element_type=f32)
    os = jnp.dot(h1, WselfT[...], preferred_element_type=f32) + bself[...]
    h2 = _gate_mix(oc, od, os, gWc[...], gWd[...], gWs[...], gb[...]) + h1
    out_ref[...] = jnp.dot(h2, clsWT[...], preferred_element_type=f32) + clsb[...]


def _tc2(conc, disc, h1, WconT, WdisT, WselfT, bself, gWc, gWd, gWs, gb,
         clsWT, clsb):
    return pl.pallas_call(
        _tc2_body,
        grid=_GRID,
        in_specs=[_rows(H), _rows(H), _rows(H), _rep((H, H)), _rep((H, H)),
                  _rep((H, H)), _rep((1, H)), _rep((H, 3)), _rep((H, 3)),
                  _rep((H, 3)), _rep((1, 3)), _rep((H, C)), _rep((1, C))],
        out_specs=[_rows(C)],
        out_shape=[jax.ShapeDtypeStruct((N, C), f32)],
    )(conc, disc, h1, WconT, WdisT, WselfT, bself, gWc, gWd, gWs, gb, clsWT, clsb)[0]


def _conv_layer(h, xg, a, b2, nrm, rowp, colp):
    xgp = jnp.pad(xg, ((0, NP - N), (0, 0)))
    ap = jnp.pad(a.reshape(-1), (0, NP - N))
    bp = jnp.pad(b2.reshape(-1), (0, NP - N))
    nrmp = jnp.pad(nrm.reshape(-1), (0, NP - N))
    ec, ed, scp, sdp = _edge_kernel(xgp, ap, bp, nrmp, rowp, colp)
    rc, rd = _recip(scp, sdp)
    hp = jnp.pad(h, ((0, NP - N), (0, 0)))
    accc, accd = _agg_kernel(hp, rowp, colp, ec, ed, rc, rd)
    return accc[:N], accd[:N]


def kernel(x, edge_index, mlp_W, mlp_b, mlp_bn_g, mlp_bn_b, mlp_bn_m, mlp_bn_v,
           c1_Wg, c1_Wh, c1_bh, c1_Wcon, c1_Wdis, c1_Wself, c1_bself, c1_gateW, c1_gateb,
           bn1_g, bn1_b, bn1_m, bn1_v,
           c2_Wg, c2_Wh, c2_bh, c2_Wcon, c2_Wdis, c2_Wself, c2_bself, c2_gateW, c2_gateb,
           cls_W, cls_b):
    loops = jnp.arange(N, dtype=edge_index.dtype)
    pad_n = ET_PAD - ET
    pad_idx = (jnp.arange(pad_n, dtype=edge_index.dtype) % 16) + N
    rowp = jnp.concatenate([edge_index[0], loops, pad_idx])
    colp = jnp.concatenate([edge_index[1], loops, pad_idx])

    r1 = lambda v: v.reshape(1, -1).astype(f32)

    h0, xg1, a1, b1, nrm1 = _tc0(
        x, mlp_W.T, r1(mlp_b), r1(mlp_bn_g), r1(mlp_bn_b), r1(mlp_bn_m),
        r1(mlp_bn_v), c1_Wg.T, c1_Wh[0, :H].reshape(H, 1),
        c1_Wh[0, H:].reshape(H, 1), c1_bh.reshape(1, 1))

    conc, disc = _conv_layer(h0, xg1, a1, b1, nrm1, rowp, colp)

    g1 = c1_gateW.T
    h1, xg2, a2, b2, nrm2 = _tc1(
        conc, disc, h0, c1_Wcon.T, c1_Wdis.T, c1_Wself.T, r1(c1_bself),
        g1[:H], g1[H:2 * H], g1[2 * H:], r1(c1_gateb),
        r1(bn1_g), r1(bn1_b), r1(bn1_m), r1(bn1_v),
        c2_Wg.T, c2_Wh[0, :H].reshape(H, 1), c2_Wh[0, H:].reshape(H, 1),
        c2_bh.reshape(1, 1))

    conc2, disc2 = _conv_layer(h1, xg2, a2, b2, nrm2, rowp, colp)

    g2 = c2_gateW.T
    return _tc2(conc2, disc2, h1, c2_Wcon.T, c2_Wdis.T, c2_Wself.T,
                r1(c2_bself), g2[:H], g2[H:2 * H], g2[2 * H:], r1(c2_gateb),
                cls_W.T, r1(cls_b))
